# Initial kernel scaffold; baseline (speedup 1.0000x reference)
import functools

import jax
import jax.numpy as jnp
from jax import lax
from jax.experimental import pallas as pl
from jax.experimental.pallas import tpu as pltpu
from jax.experimental.pallas import tpu_sc as plsc

N_NODES = 10000
N_EDGES = 320000
D_IN = 128
D_H = 128
D_CAT = 256
D_MID = 320

NC = 2
NS = 16
NW = NC * NS
EPW = N_EDGES // NW

LANES = 16

GC = 80
SCC = 80
EC = 400

NB = 1000
EB = 4000


def _node1_body(x_ref, win_ref, scale_ref, c0_ref, wdh_ref, wdx_ref,
                wbh_ref, wbx_ref, b1_ref, a_ref, b_ref):
    xb = x_ref[...]
    h = jnp.tanh(
        jnp.dot(xb, win_ref[...], preferred_element_type=jnp.float32)
        * scale_ref[...] + c0_ref[...])
    a_ref[...] = (jnp.dot(h, wdh_ref[...], preferred_element_type=jnp.float32)
                  + jnp.dot(xb, wdx_ref[...], preferred_element_type=jnp.float32)
                  + b1_ref[...])
    b_ref[...] = (jnp.dot(h, wbh_ref[...], preferred_element_type=jnp.float32)
                  + jnp.dot(xb, wbx_ref[...], preferred_element_type=jnp.float32))


def _node1(x, W_in, scale, c0, Wd_h, Wd_x, Wb_h, Wb_x, b1):
    grid = (N_NODES // NB,)
    full = lambda shape: pl.BlockSpec(shape, lambda i: (0, 0))
    return pl.pallas_call(
        _node1_body,
        grid=grid,
        in_specs=[
            pl.BlockSpec((NB, D_IN), lambda i: (i, 0)),
            full((D_IN, D_H)),
            full((1, D_H)),
            full((1, D_H)),
            full((D_H, D_MID)),
            full((D_IN, D_MID)),
            full((D_H, D_MID)),
            full((D_IN, D_MID)),
            full((1, D_MID)),
        ],
        out_specs=[
            pl.BlockSpec((NB, D_MID), lambda i: (i, 0)),
            pl.BlockSpec((NB, D_MID), lambda i: (i, 0)),
        ],
        out_shape=[
            jax.ShapeDtypeStruct((N_NODES, D_MID), jnp.float32),
            jax.ShapeDtypeStruct((N_NODES, D_MID), jnp.float32),
        ],
    )(x, W_in, scale, c0, Wd_h, Wd_x, Wb_h, Wb_x, b1)


_sc_mesh = plsc.VectorSubcoreMesh(core_axis_name="c", subcore_axis_name="s")


@functools.partial(
    pl.kernel,
    out_type=jax.ShapeDtypeStruct((N_EDGES, D_MID), jnp.float32),
    mesh=_sc_mesh,
    scratch_types=[
        pltpu.VMEM((GC,), jnp.int32),
        pltpu.VMEM((GC,), jnp.int32),
        pltpu.VMEM((GC, D_MID), jnp.float32),
        pltpu.VMEM((GC, D_MID), jnp.float32),
        pltpu.SemaphoreType.DMA,
        pltpu.SemaphoreType.DMA,
    ],
)
def _gather_add(a_hbm, b_hbm, col_hbm, row_hbm, t_hbm,
                idxc_v, idxr_v, bufa_v, bufb_v, sema, semb):
    wid = lax.axis_index("s") * NC + lax.axis_index("c")
    base = wid * EPW

    def chunk(k, carry):
        eb = base + k * GC
        pltpu.sync_copy(col_hbm.at[pl.ds(eb, GC)], idxc_v)
        pltpu.sync_copy(row_hbm.at[pl.ds(eb, GC)], idxr_v)
        ca = pltpu.async_copy(a_hbm.at[idxc_v], bufa_v, sema)
        cb = pltpu.async_copy(b_hbm.at[idxr_v], bufb_v, semb)
        ca.wait()
        cb.wait()

        def rowf(r, c2):
            for c in range(D_MID // LANES):
                s = pl.ds(c * LANES, LANES)
                bufa_v[r, s] = bufa_v[r, s] + bufb_v[r, s]
            return c2

        lax.fori_loop(0, GC, rowf, 0)
        pltpu.sync_copy(bufa_v, t_hbm.at[pl.ds(eb, GC)])
        return carry

    lax.fori_loop(0, EPW // GC, chunk, 0)


def _edge_mlp_body(t_ref, w2_ref, b2_ref, m_ref):
    t = jnp.maximum(t_ref[...], 0.0)
    m_ref[...] = jnp.maximum(
        jnp.dot(t, w2_ref[...], preferred_element_type=jnp.float32)
        + b2_ref[...], 0.0)


def _edge_mlp(t, W2, b2):
    return pl.pallas_call(
        _edge_mlp_body,
        grid=(N_EDGES // EB,),
        in_specs=[
            pl.BlockSpec((EB, D_MID), lambda i: (i, 0)),
            pl.BlockSpec((D_MID, D_H), lambda i: (0, 0)),
            pl.BlockSpec((1, D_H), lambda i: (0, 0)),
        ],
        out_specs=pl.BlockSpec((EB, D_H), lambda i: (i, 0)),
        out_shape=jax.ShapeDtypeStruct((N_EDGES, D_H), jnp.float32),
    )(t, W2, b2)


_ROWS_PER_TILE = N_NODES // NS
_EPC = N_EDGES // NC
_EPT = _EPC // NS


@functools.partial(
    pl.kernel,
    out_type=jax.ShapeDtypeStruct((NC, N_NODES, D_H), jnp.float32),
    mesh=_sc_mesh,
    scratch_types=[
        pltpu.VMEM((SCC,), jnp.int32),
        pltpu.VMEM((SCC, D_H), jnp.float32),
        pltpu.VMEM_SHARED((N_NODES, D_H), jnp.float32),
    ],
)
def _scatter_add(m_hbm, col_hbm, zeros_hbm, out_hbm, idx_v, bufm_v, acc_sh):
    cid = lax.axis_index("c")
    sid = lax.axis_index("s")
    rb = sid * _ROWS_PER_TILE
    pltpu.sync_copy(zeros_hbm.at[pl.ds(rb, _ROWS_PER_TILE)],
                    acc_sh.at[pl.ds(rb, _ROWS_PER_TILE)])
    plsc.subcore_barrier()

    base = cid * _EPC + sid * _EPT

    def chunk(k, carry):
        eb = base + k * SCC
        pltpu.sync_copy(col_hbm.at[pl.ds(eb, SCC)], idx_v)
        pltpu.sync_copy(m_hbm.at[pl.ds(eb, SCC)], bufm_v)
        pltpu.sync_copy(bufm_v, acc_sh.at[idx_v], add=True)
        return carry

    lax.fori_loop(0, _EPT // SCC, chunk, 0)
    plsc.subcore_barrier()
    pltpu.sync_copy(acc_sh.at[pl.ds(rb, _ROWS_PER_TILE)],
                    out_hbm.at[cid, pl.ds(rb, _ROWS_PER_TILE)])


def _node2_body(p_ref, x_ref, wuv1_ref, wuv2_ref, buv_ref, uv_ref):
    hn = p_ref[0] + p_ref[1]
    uv_ref[...] = (
        jnp.dot(hn, wuv1_ref[...], preferred_element_type=jnp.float32)
        + jnp.dot(x_ref[...], wuv2_ref[...], preferred_element_type=jnp.float32)
        + buv_ref[...])


def _node2(P, x, wuv1, wuv2, buv):
    return pl.pallas_call(
        _node2_body,
        grid=(N_NODES // NB,),
        in_specs=[
            pl.BlockSpec((NC, NB, D_H), lambda i: (0, i, 0)),
            pl.BlockSpec((NB, D_IN), lambda i: (i, 0)),
            pl.BlockSpec((D_H, 2), lambda i: (0, 0)),
            pl.BlockSpec((D_IN, 2), lambda i: (0, 0)),
            pl.BlockSpec((1, 2), lambda i: (0, 0)),
        ],
        out_specs=pl.BlockSpec((NB, 2), lambda i: (i, 0)),
        out_shape=jax.ShapeDtypeStruct((N_NODES, 2), jnp.float32),
    )(P, x, wuv1, wuv2, buv)


@functools.partial(
    pl.kernel,
    out_type=jax.ShapeDtypeStruct((N_EDGES,), jnp.float32),
    mesh=_sc_mesh,
    scratch_types=[
        pltpu.VMEM((N_NODES,), jnp.float32),
        pltpu.VMEM((N_NODES,), jnp.float32),
        pltpu.VMEM((EC,), jnp.int32),
        pltpu.VMEM((EC,), jnp.int32),
        pltpu.VMEM((EC,), jnp.float32),
    ],
)
def _edge_score(u_hbm, v_hbm, row_hbm, col_hbm, out_hbm,
                u_v, v_v, idxr_v, idxc_v, outb_v):
    wid = lax.axis_index("s") * NC + lax.axis_index("c")
    base = wid * EPW
    pltpu.sync_copy(u_hbm, u_v)
    pltpu.sync_copy(v_hbm, v_v)

    def chunk(k, carry):
        eb = base + k * EC
        pltpu.sync_copy(row_hbm.at[pl.ds(eb, EC)], idxr_v)
        pltpu.sync_copy(col_hbm.at[pl.ds(eb, EC)], idxc_v)

        def sub(j, c2):
            s = pl.ds(j * LANES, LANES)
            ur = plsc.load_gather(u_v, [idxr_v[s]])
            vc = plsc.load_gather(v_v, [idxc_v[s]])
            z = ur + vc
            outb_v[s] = 1.0 / (1.0 + jnp.exp(-z))
            return c2

        lax.fori_loop(0, EC // LANES, sub, 0)
        pltpu.sync_copy(outb_v, out_hbm.at[pl.ds(eb, EC)])
        return carry

    lax.fori_loop(0, EPW // EC, chunk, 0)


def kernel(x, edge_index, W_in, b_in, bn_gamma, bn_beta, bn_mean, bn_var,
           W1, b1, W2, b2, We, be):
    eps = 1e-5
    row = edge_index[0]
    col = edge_index[1]

    scale = bn_gamma / jnp.sqrt(bn_var + eps)
    c0 = b_in * scale + (bn_beta - bn_mean * scale)

    Wt = W1[:D_CAT]
    Wb = W1[D_CAT:]
    Wd = Wt - Wb
    A, B = _node1(x, W_in, scale[None], c0[None],
                  Wd[:D_H], Wd[D_H:], Wb[:D_H], Wb[D_H:], b1[None])

    t = _gather_add(A, B, col, row)
    m = _edge_mlp(t, W2, b2[None])
    P = _scatter_add(m, col, jnp.zeros((N_NODES, D_H), jnp.float32))

    wuv1 = jnp.stack([We[:D_H, 0], We[D_CAT:D_CAT + D_H, 0]], axis=1)
    wuv2 = jnp.stack([We[D_H:D_CAT, 0], We[D_CAT + D_H:, 0]], axis=1)
    buv = jnp.stack([be, jnp.zeros_like(be)], axis=1)
    uv = _node2(P, x, wuv1, wuv2, buv)
    u = uv[:, 0]
    v = uv[:, 1]

    return _edge_score(u, v, row, col)


# speedup vs baseline: 1.7743x; 1.7743x over previous
import functools

import jax
import jax.numpy as jnp
from jax import lax
from jax.experimental import pallas as pl
from jax.experimental.pallas import tpu as pltpu
from jax.experimental.pallas import tpu_sc as plsc

N_NODES = 10000
N_EDGES = 320000
D_IN = 128
D_H = 128
D_CAT = 256
D_EDGE = 512
D_MID = 320

NC = 2
NS = 16
NW = NC * NS
EPW = N_EDGES // NW

LANES = 16

GC = 80
SCC = 80
EC = 400

NB = 1000
EB = 2000


def _node1_body(x_ref, win_ref, scale_ref, c0_ref, xcat_ref):
    xb = x_ref[...]
    h = jnp.tanh(
        jnp.dot(xb, win_ref[...], preferred_element_type=jnp.float32)
        * scale_ref[...] + c0_ref[...])
    xcat_ref[:, :D_H] = h
    xcat_ref[:, D_H:] = xb


def _node1(x, W_in, scale, c0):
    return pl.pallas_call(
        _node1_body,
        grid=(N_NODES // NB,),
        in_specs=[
            pl.BlockSpec((NB, D_IN), lambda i: (i, 0)),
            pl.BlockSpec((D_IN, D_H), lambda i: (0, 0)),
            pl.BlockSpec((1, D_H), lambda i: (0, 0)),
            pl.BlockSpec((1, D_H), lambda i: (0, 0)),
        ],
        out_specs=pl.BlockSpec((NB, D_CAT), lambda i: (i, 0)),
        out_shape=jax.ShapeDtypeStruct((N_NODES, D_CAT), jnp.float32),
    )(x, W_in, scale, c0)


_sc_mesh = plsc.VectorSubcoreMesh(core_axis_name="c", subcore_axis_name="s")


@functools.partial(
    pl.kernel,
    out_type=jax.ShapeDtypeStruct((N_EDGES, D_EDGE), jnp.float32),
    mesh=_sc_mesh,
    scratch_types=[
        pltpu.VMEM((GC,), jnp.int32),
        pltpu.VMEM((GC,), jnp.int32),
        pltpu.VMEM((GC, D_CAT), jnp.float32),
        pltpu.VMEM((GC, D_CAT), jnp.float32),
        pltpu.VMEM((GC, D_EDGE), jnp.float32),
        pltpu.SemaphoreType.DMA,
        pltpu.SemaphoreType.DMA,
    ],
)
def _gather_cat(xcat_hbm, col_hbm, row_hbm, t_hbm,
                idxc_v, idxr_v, bufi_v, bufj_v, buft_v, semi, semj):
    wid = lax.axis_index("s") * NC + lax.axis_index("c")
    base = wid * EPW

    def chunk(k, carry):
        eb = base + k * GC
        pltpu.sync_copy(col_hbm.at[pl.ds(eb, GC)], idxc_v)
        pltpu.sync_copy(row_hbm.at[pl.ds(eb, GC)], idxr_v)
        ci = pltpu.async_copy(xcat_hbm.at[idxc_v], bufi_v, semi)
        cj = pltpu.async_copy(xcat_hbm.at[idxr_v], bufj_v, semj)
        ci.wait()
        cj.wait()

        def rowf(r, c2):
            for c in range(D_CAT // LANES):
                s = pl.ds(c * LANES, LANES)
                xi = bufi_v[r, s]
                buft_v[r, s] = xi
                buft_v[r, pl.ds(D_CAT + c * LANES, LANES)] = bufj_v[r, s] - xi
            return c2

        lax.fori_loop(0, GC, rowf, 0)
        pltpu.sync_copy(buft_v, t_hbm.at[pl.ds(eb, GC)])
        return carry

    lax.fori_loop(0, EPW // GC, chunk, 0)


def _edge_mlp_body(t_ref, w1_ref, b1_ref, w2_ref, b2_ref, m_ref):
    m1 = jnp.maximum(
        jnp.dot(t_ref[...], w1_ref[...], preferred_element_type=jnp.float32)
        + b1_ref[...], 0.0)
    m_ref[...] = jnp.maximum(
        jnp.dot(m1, w2_ref[...], preferred_element_type=jnp.float32)
        + b2_ref[...], 0.0)


def _edge_mlp(t, W1, b1, W2, b2):
    return pl.pallas_call(
        _edge_mlp_body,
        grid=(N_EDGES // EB,),
        in_specs=[
            pl.BlockSpec((EB, D_EDGE), lambda i: (i, 0)),
            pl.BlockSpec((D_EDGE, D_MID), lambda i: (0, 0)),
            pl.BlockSpec((1, D_MID), lambda i: (0, 0)),
            pl.BlockSpec((D_MID, D_H), lambda i: (0, 0)),
            pl.BlockSpec((1, D_H), lambda i: (0, 0)),
        ],
        out_specs=pl.BlockSpec((EB, D_H), lambda i: (i, 0)),
        out_shape=jax.ShapeDtypeStruct((N_EDGES, D_H), jnp.float32),
    )(t, W1, b1, W2, b2)


N_PAD = 10240
_ROWS_PER_TILE = N_PAD // NS
_EPC = N_EDGES // NC
_EPT = _EPC // NS


@functools.partial(
    pl.kernel,
    out_type=jax.ShapeDtypeStruct((NC, N_PAD, D_H), jnp.float32),
    mesh=_sc_mesh,
    scratch_types=[
        pltpu.VMEM((SCC,), jnp.int32),
        pltpu.VMEM((SCC, D_H), jnp.float32),
        pltpu.VMEM_SHARED((N_PAD, D_H), jnp.float32),
    ],
)
def _scatter_add(m_hbm, col_hbm, zeros_hbm, out_hbm, idx_v, bufm_v, acc_sh):
    cid = lax.axis_index("c")
    sid = lax.axis_index("s")
    rb = sid * _ROWS_PER_TILE
    pltpu.sync_copy(zeros_hbm.at[pl.ds(rb, _ROWS_PER_TILE)],
                    acc_sh.at[pl.ds(rb, _ROWS_PER_TILE)])
    plsc.subcore_barrier()

    base = cid * _EPC + sid * _EPT

    def chunk(k, carry):
        eb = base + k * SCC
        pltpu.sync_copy(col_hbm.at[pl.ds(eb, SCC)], idx_v)
        pltpu.sync_copy(m_hbm.at[pl.ds(eb, SCC)], bufm_v)
        pltpu.sync_copy(bufm_v, acc_sh.at[idx_v], add=True)
        return carry

    lax.fori_loop(0, _EPT // SCC, chunk, 0)
    plsc.subcore_barrier()
    pltpu.sync_copy(acc_sh.at[pl.ds(rb, _ROWS_PER_TILE)],
                    out_hbm.at[cid, pl.ds(rb, _ROWS_PER_TILE)])


def _node2_body(p_ref, x_ref, wuv1_ref, wuv2_ref, buv_ref, uv_ref):
    hn = p_ref[0] + p_ref[1]
    uv_ref[...] = (
        jnp.dot(hn, wuv1_ref[...], preferred_element_type=jnp.float32)
        + jnp.dot(x_ref[...], wuv2_ref[...], preferred_element_type=jnp.float32)
        + buv_ref[...])


def _node2(P, x, wuv1, wuv2, buv):
    return pl.pallas_call(
        _node2_body,
        grid=(N_NODES // NB,),
        in_specs=[
            pl.BlockSpec((NC, NB, D_H), lambda i: (0, i, 0)),
            pl.BlockSpec((NB, D_IN), lambda i: (i, 0)),
            pl.BlockSpec((D_H, 2), lambda i: (0, 0)),
            pl.BlockSpec((D_IN, 2), lambda i: (0, 0)),
            pl.BlockSpec((1, 2), lambda i: (0, 0)),
        ],
        out_specs=pl.BlockSpec((NB, 2), lambda i: (i, 0)),
        out_shape=jax.ShapeDtypeStruct((N_NODES, 2), jnp.float32),
    )(P, x, wuv1, wuv2, buv)


@functools.partial(
    pl.kernel,
    out_type=jax.ShapeDtypeStruct((N_EDGES,), jnp.float32),
    mesh=_sc_mesh,
    compiler_params=pltpu.CompilerParams(needs_layout_passes=False),
    scratch_types=[
        pltpu.VMEM((N_NODES,), jnp.float32),
        pltpu.VMEM((N_NODES,), jnp.float32),
        pltpu.VMEM((EC,), jnp.int32),
        pltpu.VMEM((EC,), jnp.int32),
        pltpu.VMEM((EC,), jnp.float32),
    ],
)
def _edge_score(u_hbm, v_hbm, row_hbm, col_hbm, out_hbm,
                u_v, v_v, idxr_v, idxc_v, outb_v):
    wid = lax.axis_index("s") * NC + lax.axis_index("c")
    base = wid * EPW
    pltpu.sync_copy(u_hbm, u_v)
    pltpu.sync_copy(v_hbm, v_v)

    def chunk(k, carry):
        eb = base + k * EC
        pltpu.sync_copy(row_hbm.at[pl.ds(eb, EC)], idxr_v)
        pltpu.sync_copy(col_hbm.at[pl.ds(eb, EC)], idxc_v)

        def sub(j, c2):
            s = pl.ds(j * LANES, LANES)
            ur = plsc.load_gather(u_v, [idxr_v[s]])
            vc = plsc.load_gather(v_v, [idxc_v[s]])
            z = ur + vc
            outb_v[s] = 1.0 / (1.0 + jnp.exp(-z))
            return c2

        lax.fori_loop(0, EC // LANES, sub, 0)
        pltpu.sync_copy(outb_v, out_hbm.at[pl.ds(eb, EC)])
        return carry

    lax.fori_loop(0, EPW // EC, chunk, 0)


def kernel(x, edge_index, W_in, b_in, bn_gamma, bn_beta, bn_mean, bn_var,
           W1, b1, W2, b2, We, be):
    eps = 1e-5
    row = edge_index[0]
    col = edge_index[1]

    scale = bn_gamma / jnp.sqrt(bn_var + eps)
    c0 = b_in * scale + (bn_beta - bn_mean * scale)

    xcat = _node1(x, W_in, scale[None], c0[None])
    t0 = _gather_cat(xcat, col, row)
    m = _edge_mlp(t0, W1, b1[None], W2, b2[None])
    P = _scatter_add(m, col, jnp.zeros((N_PAD, D_H), jnp.float32))

    wuv1 = jnp.stack([We[:D_H, 0], We[D_CAT:D_CAT + D_H, 0]], axis=1)
    wuv2 = jnp.stack([We[D_H:D_CAT, 0], We[D_CAT + D_H:, 0]], axis=1)
    buv = jnp.stack([be, jnp.zeros_like(be)], axis=1)
    uv = _node2(P, x, wuv1, wuv2, buv)
    u = uv[:, 0]
    v = uv[:, 1]

    return _edge_score(u, v, row, col)


# speedup vs baseline: 3.8886x; 2.1916x over previous
import functools

import jax
import jax.numpy as jnp
from jax import lax
from jax.experimental import pallas as pl
from jax.experimental.pallas import tpu as pltpu
from jax.experimental.pallas import tpu_sc as plsc

N_NODES = 10000
N_EDGES = 320000
D_IN = 128
D_H = 128
D_CAT = 256
D_EDGE = 512
D_MID = 320

NC = 2
NS = 16
NW = NC * NS
EPW = N_EDGES // NW

LANES = 16

GC = 80
SCC = 80
EC = 400

NB = 1000
EB = 2000


def _node1_body(x_ref, win_ref, scale_ref, c0_ref, xcat_ref):
    xb = x_ref[...]
    h = jnp.tanh(
        jnp.dot(xb, win_ref[...], preferred_element_type=jnp.float32)
        * scale_ref[...] + c0_ref[...])
    xcat_ref[:, :D_H] = h
    xcat_ref[:, D_H:] = xb


def _node1(x, W_in, scale, c0):
    return pl.pallas_call(
        _node1_body,
        grid=(N_NODES // NB,),
        in_specs=[
            pl.BlockSpec((NB, D_IN), lambda i: (i, 0)),
            pl.BlockSpec((D_IN, D_H), lambda i: (0, 0)),
            pl.BlockSpec((1, D_H), lambda i: (0, 0)),
            pl.BlockSpec((1, D_H), lambda i: (0, 0)),
        ],
        out_specs=pl.BlockSpec((NB, D_CAT), lambda i: (i, 0)),
        out_shape=jax.ShapeDtypeStruct((N_NODES, D_CAT), jnp.float32),
    )(x, W_in, scale, c0)


_sc_mesh = plsc.VectorSubcoreMesh(core_axis_name="c", subcore_axis_name="s")


@functools.partial(
    pl.kernel,
    out_type=[
        jax.ShapeDtypeStruct((N_EDGES, D_CAT), jnp.float32),
        jax.ShapeDtypeStruct((N_EDGES, D_CAT), jnp.float32),
    ],
    mesh=_sc_mesh,
    scratch_types=[
        pltpu.VMEM((EPW,), jnp.int32),
        pltpu.VMEM((EPW,), jnp.int32),
        pltpu.VMEM((2, GC, D_CAT), jnp.float32),
        pltpu.VMEM((2, GC, D_CAT), jnp.float32),
        pltpu.SemaphoreType.DMA,
        pltpu.SemaphoreType.DMA,
        pltpu.SemaphoreType.DMA,
        pltpu.SemaphoreType.DMA,
    ],
)
def _gather_cat(xcat_hbm, col_hbm, row_hbm, gxi_hbm, gxj_hbm,
                idxc_v, idxr_v, bufi_v, bufj_v, semg0, semg1, semo0, semo1):
    wid = lax.axis_index("s") * NC + lax.axis_index("c")
    base = wid * EPW
    nchunk = EPW // GC
    semg = (semg0, semg1)
    semo = (semo0, semo1)

    pltpu.sync_copy(col_hbm.at[pl.ds(base, EPW)], idxc_v)
    pltpu.sync_copy(row_hbm.at[pl.ds(base, EPW)], idxr_v)

    def gather_start(k, slot):
        pltpu.async_copy(xcat_hbm.at[idxc_v.at[pl.ds(k * GC, GC)]],
                         bufi_v.at[slot], semg[slot])
        pltpu.async_copy(xcat_hbm.at[idxr_v.at[pl.ds(k * GC, GC)]],
                         bufj_v.at[slot], semg[slot])

    def gather_wait(slot):
        pltpu.make_async_copy(xcat_hbm.at[pl.ds(0, GC)], bufi_v.at[slot], semg[slot]).wait()
        pltpu.make_async_copy(xcat_hbm.at[pl.ds(0, GC)], bufj_v.at[slot], semg[slot]).wait()

    def out_start(k, slot):
        eb = base + k * GC
        pltpu.async_copy(bufi_v.at[slot], gxi_hbm.at[pl.ds(eb, GC)], semo[slot])
        pltpu.async_copy(bufj_v.at[slot], gxj_hbm.at[pl.ds(eb, GC)], semo[slot])

    def out_wait(slot):
        pltpu.make_async_copy(gxi_hbm.at[pl.ds(0, GC)], bufi_v.at[slot], semo[slot]).wait()
        pltpu.make_async_copy(gxi_hbm.at[pl.ds(0, GC)], bufj_v.at[slot], semo[slot]).wait()

    gather_start(0, 0)

    def pair(g, carry):
        for b in range(2):
            k = g * 2 + b
            @pl.when(k > 0)
            def _():
                out_wait(1 - b)

            gather_start(k + 1, 1 - b)
            gather_wait(b)
            out_start(k, b)
        return carry

    lax.fori_loop(0, (nchunk - 1) // 2, pair, 0)
    gather_wait(0)
    out_start(nchunk - 1, 0)
    out_wait(1)
    out_wait(0)


def _edge_mlp_body(xi_ref, xj_ref, w1t_ref, w1b_ref, b1_ref, w2_ref, b2_ref, m_ref):
    xi = xi_ref[...]
    d = xj_ref[...] - xi
    m1 = jnp.maximum(
        jnp.dot(xi, w1t_ref[...], preferred_element_type=jnp.float32)
        + jnp.dot(d, w1b_ref[...], preferred_element_type=jnp.float32)
        + b1_ref[...], 0.0)
    m_ref[...] = jnp.maximum(
        jnp.dot(m1, w2_ref[...], preferred_element_type=jnp.float32)
        + b2_ref[...], 0.0)


def _edge_mlp(gxi, gxj, W1t, W1b, b1, W2, b2):
    return pl.pallas_call(
        _edge_mlp_body,
        grid=(N_EDGES // EB,),
        in_specs=[
            pl.BlockSpec((EB, D_CAT), lambda i: (i, 0)),
            pl.BlockSpec((EB, D_CAT), lambda i: (i, 0)),
            pl.BlockSpec((D_CAT, D_MID), lambda i: (0, 0)),
            pl.BlockSpec((D_CAT, D_MID), lambda i: (0, 0)),
            pl.BlockSpec((1, D_MID), lambda i: (0, 0)),
            pl.BlockSpec((D_MID, D_H), lambda i: (0, 0)),
            pl.BlockSpec((1, D_H), lambda i: (0, 0)),
        ],
        out_specs=pl.BlockSpec((EB, D_H), lambda i: (i, 0)),
        out_shape=jax.ShapeDtypeStruct((N_EDGES, D_H), jnp.float32),
    )(gxi, gxj, W1t, W1b, b1, W2, b2)


N_PAD = 10240
_ROWS_PER_TILE = N_PAD // NS
_EPC = N_EDGES // NC
_EPT = _EPC // NS


@functools.partial(
    pl.kernel,
    out_type=jax.ShapeDtypeStruct((NC, N_PAD, D_H), jnp.float32),
    mesh=_sc_mesh,
    scratch_types=[
        pltpu.VMEM((SCC,), jnp.int32),
        pltpu.VMEM((SCC, D_H), jnp.float32),
        pltpu.VMEM_SHARED((N_PAD, D_H), jnp.float32),
    ],
)
def _scatter_add(m_hbm, col_hbm, zeros_hbm, out_hbm, idx_v, bufm_v, acc_sh):
    cid = lax.axis_index("c")
    sid = lax.axis_index("s")
    rb = sid * _ROWS_PER_TILE
    pltpu.sync_copy(zeros_hbm.at[pl.ds(rb, _ROWS_PER_TILE)],
                    acc_sh.at[pl.ds(rb, _ROWS_PER_TILE)])
    plsc.subcore_barrier()

    base = cid * _EPC + sid * _EPT

    def chunk(k, carry):
        eb = base + k * SCC
        pltpu.sync_copy(col_hbm.at[pl.ds(eb, SCC)], idx_v)
        pltpu.sync_copy(m_hbm.at[pl.ds(eb, SCC)], bufm_v)
        pltpu.sync_copy(bufm_v, acc_sh.at[idx_v], add=True)
        return carry

    lax.fori_loop(0, _EPT // SCC, chunk, 0)
    plsc.subcore_barrier()
    pltpu.sync_copy(acc_sh.at[pl.ds(rb, _ROWS_PER_TILE)],
                    out_hbm.at[cid, pl.ds(rb, _ROWS_PER_TILE)])


def _node2_body(p_ref, x_ref, wuv1_ref, wuv2_ref, buv_ref, uv_ref):
    hn = p_ref[0] + p_ref[1]
    uv_ref[...] = (
        jnp.dot(hn, wuv1_ref[...], preferred_element_type=jnp.float32)
        + jnp.dot(x_ref[...], wuv2_ref[...], preferred_element_type=jnp.float32)
        + buv_ref[...])


def _node2(P, x, wuv1, wuv2, buv):
    return pl.pallas_call(
        _node2_body,
        grid=(N_NODES // NB,),
        in_specs=[
            pl.BlockSpec((NC, NB, D_H), lambda i: (0, i, 0)),
            pl.BlockSpec((NB, D_IN), lambda i: (i, 0)),
            pl.BlockSpec((D_H, 2), lambda i: (0, 0)),
            pl.BlockSpec((D_IN, 2), lambda i: (0, 0)),
            pl.BlockSpec((1, 2), lambda i: (0, 0)),
        ],
        out_specs=pl.BlockSpec((NB, 2), lambda i: (i, 0)),
        out_shape=jax.ShapeDtypeStruct((N_NODES, 2), jnp.float32),
    )(P, x, wuv1, wuv2, buv)


@functools.partial(
    pl.kernel,
    out_type=jax.ShapeDtypeStruct((N_EDGES,), jnp.float32),
    mesh=_sc_mesh,
    compiler_params=pltpu.CompilerParams(needs_layout_passes=False),
    scratch_types=[
        pltpu.VMEM((N_NODES,), jnp.float32),
        pltpu.VMEM((N_NODES,), jnp.float32),
        pltpu.VMEM((EC,), jnp.int32),
        pltpu.VMEM((EC,), jnp.int32),
        pltpu.VMEM((EC,), jnp.float32),
    ],
)
def _edge_score(u_hbm, v_hbm, row_hbm, col_hbm, out_hbm,
                u_v, v_v, idxr_v, idxc_v, outb_v):
    wid = lax.axis_index("s") * NC + lax.axis_index("c")
    base = wid * EPW
    pltpu.sync_copy(u_hbm, u_v)
    pltpu.sync_copy(v_hbm, v_v)

    def chunk(k, carry):
        eb = base + k * EC
        pltpu.sync_copy(row_hbm.at[pl.ds(eb, EC)], idxr_v)
        pltpu.sync_copy(col_hbm.at[pl.ds(eb, EC)], idxc_v)

        def sub(j, c2):
            s = pl.ds(j * LANES, LANES)
            ur = plsc.load_gather(u_v, [idxr_v[s]])
            vc = plsc.load_gather(v_v, [idxc_v[s]])
            z = ur + vc
            outb_v[s] = 1.0 / (1.0 + jnp.exp(-z))
            return c2

        lax.fori_loop(0, EC // LANES, sub, 0)
        pltpu.sync_copy(outb_v, out_hbm.at[pl.ds(eb, EC)])
        return carry

    lax.fori_loop(0, EPW // EC, chunk, 0)


def kernel(x, edge_index, W_in, b_in, bn_gamma, bn_beta, bn_mean, bn_var,
           W1, b1, W2, b2, We, be):
    eps = 1e-5
    row = edge_index[0]
    col = edge_index[1]

    scale = bn_gamma / jnp.sqrt(bn_var + eps)
    c0 = b_in * scale + (bn_beta - bn_mean * scale)

    xcat = _node1(x, W_in, scale[None], c0[None])
    gxi, gxj = _gather_cat(xcat, col, row)
    m = _edge_mlp(gxi, gxj, W1[:D_CAT], W1[D_CAT:], b1[None], W2, b2[None])
    P = _scatter_add(m, col, jnp.zeros((N_PAD, D_H), jnp.float32))

    wuv1 = jnp.stack([We[:D_H, 0], We[D_CAT:D_CAT + D_H, 0]], axis=1)
    wuv2 = jnp.stack([We[D_H:D_CAT, 0], We[D_CAT + D_H:, 0]], axis=1)
    buv = jnp.stack([be, jnp.zeros_like(be)], axis=1)
    uv = _node2(P, x, wuv1, wuv2, buv)
    u = uv[:, 0]
    v = uv[:, 1]

    return _edge_score(u, v, row, col)


# speedup vs baseline: 4.0586x; 1.0437x over previous
import functools

import jax
import jax.numpy as jnp
from jax import lax
from jax.experimental import pallas as pl
from jax.experimental.pallas import tpu as pltpu
from jax.experimental.pallas import tpu_sc as plsc

N_NODES = 10000
N_EDGES = 320000
D_IN = 128
D_H = 128
D_CAT = 256
D_EDGE = 512
D_MID = 320

NC = 2
NS = 16
NW = NC * NS
EPW = N_EDGES // NW

LANES = 16

GC = 80
SCC = 80
EC = 400

NB = 1000
EB = 2000


def _node1_body(x_ref, win_ref, scale_ref, c0_ref, xcat_ref):
    xb = x_ref[...]
    h = jnp.tanh(
        jnp.dot(xb, win_ref[...], preferred_element_type=jnp.float32)
        * scale_ref[...] + c0_ref[...])
    xcat_ref[:, :D_H] = h
    xcat_ref[:, D_H:] = xb


def _node1(x, W_in, scale, c0):
    return pl.pallas_call(
        _node1_body,
        grid=(N_NODES // NB,),
        in_specs=[
            pl.BlockSpec((NB, D_IN), lambda i: (i, 0)),
            pl.BlockSpec((D_IN, D_H), lambda i: (0, 0)),
            pl.BlockSpec((1, D_H), lambda i: (0, 0)),
            pl.BlockSpec((1, D_H), lambda i: (0, 0)),
        ],
        out_specs=pl.BlockSpec((NB, D_CAT), lambda i: (i, 0)),
        out_shape=jax.ShapeDtypeStruct((N_NODES, D_CAT), jnp.float32),
    )(x, W_in, scale, c0)


_sc_mesh = plsc.VectorSubcoreMesh(core_axis_name="c", subcore_axis_name="s")


E_SLAB = N_EDGES // 2
EPW_S = E_SLAB // NW
GCS = 40


@functools.partial(
    pl.kernel,
    out_type=[
        jax.ShapeDtypeStruct((E_SLAB, D_CAT), jnp.float32),
        jax.ShapeDtypeStruct((E_SLAB, D_CAT), jnp.float32),
    ],
    mesh=_sc_mesh,
    scratch_types=[
        pltpu.VMEM((EPW_S,), jnp.int32),
        pltpu.VMEM((EPW_S,), jnp.int32),
        pltpu.VMEM((2, GCS, D_CAT), jnp.float32),
        pltpu.VMEM((2, GCS, D_CAT), jnp.float32),
        pltpu.SemaphoreType.DMA,
        pltpu.SemaphoreType.DMA,
        pltpu.SemaphoreType.DMA,
        pltpu.SemaphoreType.DMA,
    ],
)
def _gather_cat(xcat_hbm, col_hbm, row_hbm, gxi_hbm, gxj_hbm,
                idxc_v, idxr_v, bufi_v, bufj_v, semg0, semg1, semo0, semo1):
    wid = lax.axis_index("s") * NC + lax.axis_index("c")
    base = wid * EPW_S
    nchunk = EPW_S // GCS
    semg = (semg0, semg1)
    semo = (semo0, semo1)

    pltpu.sync_copy(col_hbm.at[pl.ds(base, EPW_S)], idxc_v)
    pltpu.sync_copy(row_hbm.at[pl.ds(base, EPW_S)], idxr_v)

    def gather_start(k, slot):
        pltpu.async_copy(xcat_hbm.at[idxc_v.at[pl.ds(k * GCS, GCS)]],
                         bufi_v.at[slot], semg[slot])
        pltpu.async_copy(xcat_hbm.at[idxr_v.at[pl.ds(k * GCS, GCS)]],
                         bufj_v.at[slot], semg[slot])

    def gather_wait(slot):
        pltpu.make_async_copy(xcat_hbm.at[pl.ds(0, GCS)], bufi_v.at[slot], semg[slot]).wait()
        pltpu.make_async_copy(xcat_hbm.at[pl.ds(0, GCS)], bufj_v.at[slot], semg[slot]).wait()

    def out_start(k, slot):
        eb = base + k * GCS
        pltpu.async_copy(bufi_v.at[slot], gxi_hbm.at[pl.ds(eb, GCS)], semo[slot])
        pltpu.async_copy(bufj_v.at[slot], gxj_hbm.at[pl.ds(eb, GCS)], semo[slot])

    def out_wait(slot):
        pltpu.make_async_copy(gxi_hbm.at[pl.ds(0, GCS)], bufi_v.at[slot], semo[slot]).wait()
        pltpu.make_async_copy(gxi_hbm.at[pl.ds(0, GCS)], bufj_v.at[slot], semo[slot]).wait()

    gather_start(0, 0)

    def pair(g, carry):
        for b in range(2):
            k = g * 2 + b
            @pl.when(k > 0)
            def _():
                out_wait(1 - b)

            gather_start(k + 1, 1 - b)
            gather_wait(b)
            out_start(k, b)
        return carry

    lax.fori_loop(0, (nchunk - 1) // 2, pair, 0)
    gather_wait(0)
    out_start(nchunk - 1, 0)
    out_wait(1)
    out_wait(0)


def _edge_mlp_body(xi_ref, xj_ref, w1t_ref, w1b_ref, b1_ref, w2_ref, b2_ref, m_ref):
    xi = xi_ref[...]
    d = xj_ref[...] - xi
    m1 = jnp.maximum(
        jnp.dot(xi, w1t_ref[...], preferred_element_type=jnp.float32)
        + jnp.dot(d, w1b_ref[...], preferred_element_type=jnp.float32)
        + b1_ref[...], 0.0)
    m_ref[...] = jnp.maximum(
        jnp.dot(m1, w2_ref[...], preferred_element_type=jnp.float32)
        + b2_ref[...], 0.0)


def _edge_mlp(gxi, gxj, W1t, W1b, b1, W2, b2):
    return pl.pallas_call(
        _edge_mlp_body,
        grid=(gxi.shape[0] // EB,),
        in_specs=[
            pl.BlockSpec((EB, D_CAT), lambda i: (i, 0)),
            pl.BlockSpec((EB, D_CAT), lambda i: (i, 0)),
            pl.BlockSpec((D_CAT, D_MID), lambda i: (0, 0)),
            pl.BlockSpec((D_CAT, D_MID), lambda i: (0, 0)),
            pl.BlockSpec((1, D_MID), lambda i: (0, 0)),
            pl.BlockSpec((D_MID, D_H), lambda i: (0, 0)),
            pl.BlockSpec((1, D_H), lambda i: (0, 0)),
        ],
        out_specs=pl.BlockSpec((EB, D_H), lambda i: (i, 0)),
        out_shape=jax.ShapeDtypeStruct((gxi.shape[0], D_H), jnp.float32),
    )(gxi, gxj, W1t, W1b, b1, W2, b2)


N_PAD = 10240
_ROWS_PER_TILE = N_PAD // NS
_EPC = N_EDGES // NC
_EPT = _EPC // NS


@functools.partial(
    pl.kernel,
    out_type=jax.ShapeDtypeStruct((NC, N_PAD, D_H), jnp.float32),
    mesh=_sc_mesh,
    scratch_types=[
        pltpu.VMEM((SCC,), jnp.int32),
        pltpu.VMEM((SCC, D_H), jnp.float32),
        pltpu.VMEM_SHARED((N_PAD, D_H), jnp.float32),
    ],
)
def _scatter_add(m0_hbm, m1_hbm, col_hbm, zeros_hbm, out_hbm, idx_v, bufm_v, acc_sh):
    cid = lax.axis_index("c")
    sid = lax.axis_index("s")
    rb = sid * _ROWS_PER_TILE
    pltpu.sync_copy(zeros_hbm.at[pl.ds(rb, _ROWS_PER_TILE)],
                    acc_sh.at[pl.ds(rb, _ROWS_PER_TILE)])
    plsc.subcore_barrier()

    def run(m_hbm):
        col_base = cid * _EPC + sid * _EPT
        m_base = sid * _EPT

        def chunk(k, carry):
            pltpu.sync_copy(col_hbm.at[pl.ds(col_base + k * SCC, SCC)], idx_v)
            pltpu.sync_copy(m_hbm.at[pl.ds(m_base + k * SCC, SCC)], bufm_v)
            pltpu.sync_copy(bufm_v, acc_sh.at[idx_v], add=True)
            return carry

        lax.fori_loop(0, _EPT // SCC, chunk, 0)

    @pl.when(cid == 0)
    def _():
        run(m0_hbm)

    @pl.when(cid == 1)
    def _():
        run(m1_hbm)

    plsc.subcore_barrier()
    pltpu.sync_copy(acc_sh.at[pl.ds(rb, _ROWS_PER_TILE)],
                    out_hbm.at[cid, pl.ds(rb, _ROWS_PER_TILE)])


def _node2_body(p_ref, x_ref, wuv1_ref, wuv2_ref, buv_ref, uv_ref):
    hn = p_ref[0] + p_ref[1]
    uv_ref[...] = (
        jnp.dot(hn, wuv1_ref[...], preferred_element_type=jnp.float32)
        + jnp.dot(x_ref[...], wuv2_ref[...], preferred_element_type=jnp.float32)
        + buv_ref[...])


def _node2(P, x, wuv1, wuv2, buv):
    return pl.pallas_call(
        _node2_body,
        grid=(N_NODES // NB,),
        in_specs=[
            pl.BlockSpec((NC, NB, D_H), lambda i: (0, i, 0)),
            pl.BlockSpec((NB, D_IN), lambda i: (i, 0)),
            pl.BlockSpec((D_H, 2), lambda i: (0, 0)),
            pl.BlockSpec((D_IN, 2), lambda i: (0, 0)),
            pl.BlockSpec((1, 2), lambda i: (0, 0)),
        ],
        out_specs=pl.BlockSpec((NB, 2), lambda i: (i, 0)),
        out_shape=jax.ShapeDtypeStruct((N_NODES, 2), jnp.float32),
    )(P, x, wuv1, wuv2, buv)


@functools.partial(
    pl.kernel,
    out_type=jax.ShapeDtypeStruct((N_EDGES,), jnp.float32),
    mesh=_sc_mesh,
    compiler_params=pltpu.CompilerParams(needs_layout_passes=False),
    scratch_types=[
        pltpu.VMEM((N_NODES,), jnp.float32),
        pltpu.VMEM((N_NODES,), jnp.float32),
        pltpu.VMEM((EC,), jnp.int32),
        pltpu.VMEM((EC,), jnp.int32),
        pltpu.VMEM((EC,), jnp.float32),
    ],
)
def _edge_score(u_hbm, v_hbm, row_hbm, col_hbm, out_hbm,
                u_v, v_v, idxr_v, idxc_v, outb_v):
    wid = lax.axis_index("s") * NC + lax.axis_index("c")
    base = wid * EPW
    pltpu.sync_copy(u_hbm, u_v)
    pltpu.sync_copy(v_hbm, v_v)

    def chunk(k, carry):
        eb = base + k * EC
        pltpu.sync_copy(row_hbm.at[pl.ds(eb, EC)], idxr_v)
        pltpu.sync_copy(col_hbm.at[pl.ds(eb, EC)], idxc_v)

        def sub(j, c2):
            s = pl.ds(j * LANES, LANES)
            ur = plsc.load_gather(u_v, [idxr_v[s]])
            vc = plsc.load_gather(v_v, [idxc_v[s]])
            z = ur + vc
            outb_v[s] = 1.0 / (1.0 + jnp.exp(-z))
            return c2

        lax.fori_loop(0, EC // LANES, sub, 0)
        pltpu.sync_copy(outb_v, out_hbm.at[pl.ds(eb, EC)])
        return carry

    lax.fori_loop(0, EPW // EC, chunk, 0)


def kernel(x, edge_index, W_in, b_in, bn_gamma, bn_beta, bn_mean, bn_var,
           W1, b1, W2, b2, We, be):
    eps = 1e-5
    row = edge_index[0]
    col = edge_index[1]

    scale = bn_gamma / jnp.sqrt(bn_var + eps)
    c0 = b_in * scale + (bn_beta - bn_mean * scale)

    xcat = _node1(x, W_in, scale[None], c0[None])
    col0, col1 = col[:E_SLAB], col[E_SLAB:]
    row0, row1 = row[:E_SLAB], row[E_SLAB:]
    gxi0, gxj0 = _gather_cat(xcat, col0, row0)
    gxi1, gxj1 = _gather_cat(xcat, col1, row1)
    W1t, W1b = W1[:D_CAT], W1[D_CAT:]
    m0 = _edge_mlp(gxi0, gxj0, W1t, W1b, b1[None], W2, b2[None])
    m1 = _edge_mlp(gxi1, gxj1, W1t, W1b, b1[None], W2, b2[None])
    P = _scatter_add(m0, m1, col, jnp.zeros((N_PAD, D_H), jnp.float32))

    wuv1 = jnp.stack([We[:D_H, 0], We[D_CAT:D_CAT + D_H, 0]], axis=1)
    wuv2 = jnp.stack([We[D_H:D_CAT, 0], We[D_CAT + D_H:, 0]], axis=1)
    buv = jnp.stack([be, jnp.zeros_like(be)], axis=1)
    uv = _node2(P, x, wuv1, wuv2, buv)
    u = uv[:, 0]
    v = uv[:, 1]

    return _edge_score(u, v, row, col)
